# baseline (speedup 1.0000x reference)
import functools

import jax
import jax.numpy as jnp
from jax import lax
from jax.experimental import pallas as pl
from jax.experimental.pallas import tpu as pltpu
from jax.experimental.pallas import tpu_sc as plsc

_NUM_CORES = 2
_NUM_SUBCORES = 16
_NUM_WORKERS = _NUM_CORES * _NUM_SUBCORES

_LANES = 16
_R = 1248
_C = 512

_PARAMS = pltpu.CompilerParams(
    use_tc_tiling_on_sc=False, needs_layout_passes=False)


def _iota16():
    return lax.iota(jnp.int32, _LANES)


def _splat16(v):
    return jnp.full((_LANES,), v, jnp.int32)


def _make_transpose_kernel(V, D):
    nsteps = -(-(V // _NUM_WORKERS + 16) // _R)
    ngroups = _R // _LANES
    mesh = plsc.VectorSubcoreMesh(core_axis_name="c", subcore_axis_name="s")

    @functools.partial(
        pl.kernel,
        mesh=mesh,
        out_type=jax.ShapeDtypeStruct((V * D,), jnp.float32),
        scratch_types=[
            pltpu.VMEM((D, _R), jnp.float32),
            pltpu.VMEM((D, _R), jnp.float32),
            pltpu.VMEM((_R * D,), jnp.float32),
            pltpu.SemaphoreType.DMA((2,)),
            pltpu.SemaphoreType.DMA,
        ],
        compiler_params=_PARAMS,
    )
    def transpose_k(wt_hbm, out_hbm, in_v0, in_v1, out_v, sem_i, sem_o):
        wid = lax.axis_index("s") * _NUM_CORES + lax.axis_index("c")
        in_refs = (in_v0, in_v1)
        lo = (wid * (V // _NUM_WORKERS)) // 8 * 8
        hi = jnp.where(wid == _NUM_WORKERS - 1, V,
                       ((wid + 1) * (V // _NUM_WORKERS)) // 8 * 8)

        def step_off(s):
            return pl.multiple_of(jnp.minimum(lo + s * _R, hi - _R), 8)

        def start_in(s, b):
            pltpu.make_async_copy(
                wt_hbm.at[:, pl.ds(step_off(s), _R)], in_refs[b], sem_i.at[b]
            ).start()

        def wait_in(b):
            pltpu.make_async_copy(
                wt_hbm.at[:, pl.ds(0, _R)], in_refs[b], sem_i.at[b]
            ).wait()

        def start_out(s):
            pltpu.make_async_copy(
                out_v, out_hbm.at[pl.ds(step_off(s) * D, _R * D)], sem_o
            ).start()

        def wait_out():
            pltpu.make_async_copy(
                out_v, out_hbm.at[pl.ds(0, _R * D)], sem_o
            ).wait()

        def transpose_step(in_ref):
            def body(g, carry):
                rvec = g * _LANES + _iota16()
                out_base = rvec * D
                for c in range(D):
                    v = plsc.load_gather(in_ref, [_splat16(c), rvec])
                    plsc.store_scatter(out_v, [out_base + c], v)
                return carry

            lax.fori_loop(0, ngroups, body, 0)

        def half_step(s, b, first):
            wait_in(b)
            if not first:
                wait_out()
            transpose_step(in_refs[b])
            start_out(s)
            start_in(s + 2, b)

        start_in(jnp.int32(0), 0)
        start_in(jnp.int32(1), 1)
        half_step(jnp.int32(0), 0, True)
        half_step(jnp.int32(1), 1, False)

        def pair_body(it, carry):
            half_step(2 * it, 0, False)
            half_step(2 * it + 1, 1, False)
            return carry

        npairs = -(-nsteps // 2)
        lax.fori_loop(1, npairs, pair_body, 0)

        wait_in(0)
        wait_in(1)
        wait_out()

    return transpose_k


def _make_gather_kernel(V, D, N, W):
    B = W * N
    per_w = B // _NUM_WORKERS
    nunits = per_w // _C
    nblocks = N // _C
    kgroups = _C // _LANES
    mesh = plsc.VectorSubcoreMesh(core_axis_name="c", subcore_axis_name="s")

    @functools.partial(
        pl.kernel,
        mesh=mesh,
        out_type=jax.ShapeDtypeStruct((W, D, N), jnp.float32),
        scratch_types=[
            pltpu.VMEM((per_w,), jnp.int32),
            pltpu.VMEM((_C, D), jnp.float32),
            pltpu.VMEM((_C, D), jnp.float32),
            pltpu.VMEM((D, _C), jnp.float32),
            pltpu.VMEM((D, _C), jnp.float32),
            pltpu.SemaphoreType.DMA((2,)),
            pltpu.SemaphoreType.DMA((2,)),
        ],
        compiler_params=_PARAMS,
    )
    def gather_k(idx_hbm, table_hbm, out_hbm, idx_v,
                 rows_v0, rows_v1, rowst_v0, rowst_v1, sem_g, sem_w):
        wid = lax.axis_index("s") * _NUM_CORES + lax.axis_index("c")
        u0 = wid * nunits
        rows_refs = (rows_v0, rows_v1)
        rowst_refs = (rowst_v0, rowst_v1)

        pltpu.sync_copy(idx_hbm.at[pl.ds(u0 * _C, per_w)], idx_v)

        def start_gather(u, b):
            um = jnp.minimum(u, nunits - 1)
            pltpu.make_async_copy(
                table_hbm.at[idx_v.at[pl.ds(um * _C, _C)]], rows_refs[b],
                sem_g.at[b],
            ).start()

        def wait_gather(b):
            pltpu.make_async_copy(
                table_hbm.at[idx_v.at[pl.ds(0, _C)]], rows_refs[b],
                sem_g.at[b],
            ).wait()

        def start_write(u, b):
            ug = u0 + u
            w = ug // nblocks
            nb = ug % nblocks
            pltpu.make_async_copy(
                rowst_refs[b],
                out_hbm.at[w, :, pl.ds(nb * _C, _C)],
                sem_w.at[b],
            ).start()

        def wait_write(b):
            pltpu.make_async_copy(
                rowst_refs[b],
                out_hbm.at[0, :, pl.ds(0, _C)],
                sem_w.at[b],
            ).wait()

        def transpose_chunk(rows_ref, rowst_ref):
            def body(k, carry):
                rvec = k * _LANES + _iota16()
                for c in range(D):
                    v = plsc.load_gather(rows_ref, [rvec, _splat16(c)])
                    plsc.store_scatter(rowst_ref, [_splat16(c), rvec], v)
                return carry

            lax.fori_loop(0, kgroups, body, 0)

        def half_step(u, b, first):
            wait_gather(b)
            if not first:
                wait_write(b)
            transpose_chunk(rows_refs[b], rowst_refs[b])
            start_write(u, b)
            start_gather(u + 2, b)

        start_gather(jnp.int32(0), 0)
        start_gather(jnp.int32(1), 1)
        half_step(jnp.int32(0), 0, True)
        half_step(jnp.int32(1), 1, True)

        def pair_body(it, carry):
            half_step(2 * it, 0, False)
            half_step(2 * it + 1, 1, False)
            return carry

        lax.fori_loop(1, nunits // 2, pair_body, 0)

        wait_gather(0)
        wait_gather(1)
        wait_write(0)
        wait_write(1)

    return gather_k


def kernel(input, weight):
    N, W = input.shape
    V, D = weight.shape
    idx = input.T.reshape(W * N).astype(jnp.int32)
    table = _make_transpose_kernel(V, D)(weight.T).reshape(V, D)
    out3 = _make_gather_kernel(V, D, N, W)(idx, table)
    return out3.transpose(2, 0, 1)


# speedup vs baseline: 4.1089x; 4.1089x over previous
import functools

import jax
import jax.numpy as jnp
from jax import lax
from jax.experimental import pallas as pl
from jax.experimental.pallas import tpu as pltpu
from jax.experimental.pallas import tpu_sc as plsc

_NUM_CORES = 2
_NUM_SUBCORES = 16
_NUM_WORKERS = _NUM_CORES * _NUM_SUBCORES


@functools.partial(jax.jit, static_argnums=(2, 3))
def _embedding_gather(idx, weight, B, D):
    b_per_w = B // _NUM_WORKERS
    C = 1600
    nchunks = b_per_w // C
    mesh = plsc.VectorSubcoreMesh(core_axis_name="c", subcore_axis_name="s")

    @functools.partial(
        pl.kernel,
        mesh=mesh,
        out_type=jax.ShapeDtypeStruct((B, D), jnp.float32),
        scratch_types=[
            pltpu.VMEM((2, C), jnp.int32),
            pltpu.VMEM((2, C, D), jnp.float32),
            pltpu.SemaphoreType.DMA((2,)),
            pltpu.SemaphoreType.DMA((2,)),
            pltpu.SemaphoreType.DMA((2,)),
        ],
        compiler_params=pltpu.CompilerParams(use_tc_tiling_on_sc=False),
    )
    def k(idx_hbm, table_hbm, out_hbm, idx_v, rows_v, sem_i, sem_g, sem_s):
        wid = lax.axis_index("s") * _NUM_CORES + lax.axis_index("c")
        base = wid * b_per_w

        idx_cp = [None] * nchunks
        gat_cp = [None] * nchunks
        st_cp = [None] * nchunks

        def start_idx(i):
            b = i % 2
            idx_cp[i] = pltpu.make_async_copy(
                idx_hbm.at[pl.ds(base + i * C, C)], idx_v.at[b], sem_i.at[b]
            )
            idx_cp[i].start()

        start_idx(0)
        for i in range(nchunks):
            b = i % 2
            if i + 1 < nchunks:
                start_idx(i + 1)
            idx_cp[i].wait()
            if i >= 2:
                st_cp[i - 2].wait()
            gat_cp[i] = pltpu.make_async_copy(
                table_hbm.at[idx_v.at[b]], rows_v.at[b], sem_g.at[b]
            )
            gat_cp[i].start()
            gat_cp[i].wait()
            st_cp[i] = pltpu.make_async_copy(
                rows_v.at[b], out_hbm.at[pl.ds(base + i * C, C)], sem_s.at[b]
            )
            st_cp[i].start()
        st_cp[nchunks - 2].wait()
        st_cp[nchunks - 1].wait()

    return k(idx, weight)


def kernel(input, weight):
    N, W = input.shape
    D = weight.shape[1]
    B = N * W
    idx = input.T.reshape(B).astype(jnp.int32)
    out = _embedding_gather(idx, weight, B, D)
    return out.reshape(W, N, D).transpose(1, 0, 2)


# speedup vs baseline: 4.1144x; 1.0013x over previous
import functools

import jax
import jax.numpy as jnp
from jax import lax
from jax.experimental import pallas as pl
from jax.experimental.pallas import tpu as pltpu
from jax.experimental.pallas import tpu_sc as plsc

_NUM_CORES = 2
_NUM_SUBCORES = 16
_NUM_WORKERS = _NUM_CORES * _NUM_SUBCORES


@functools.partial(jax.jit, static_argnums=(2, 3))
def _embedding_gather(idx, weight, B, D):
    b_per_w = B // _NUM_WORKERS
    C = 1280
    nchunks = b_per_w // C
    mesh = plsc.VectorSubcoreMesh(core_axis_name="c", subcore_axis_name="s")

    @functools.partial(
        pl.kernel,
        mesh=mesh,
        out_type=jax.ShapeDtypeStruct((B, D), jnp.float32),
        scratch_types=[
            pltpu.VMEM((2, C), jnp.int32),
            pltpu.VMEM((2, C, D), jnp.float32),
            pltpu.SemaphoreType.DMA((2,)),
            pltpu.SemaphoreType.DMA((2,)),
            pltpu.SemaphoreType.DMA((2,)),
        ],
        compiler_params=pltpu.CompilerParams(use_tc_tiling_on_sc=False),
    )
    def k(idx_hbm, table_hbm, out_hbm, idx_v, rows_v, sem_i, sem_g, sem_s):
        wid = lax.axis_index("s") * _NUM_CORES + lax.axis_index("c")
        base = wid * b_per_w

        idx_cp = [None] * nchunks
        gat_cp = [None] * nchunks
        st_cp = [None] * nchunks

        def start_idx(i):
            b = i % 2
            idx_cp[i] = pltpu.make_async_copy(
                idx_hbm.at[pl.ds(base + i * C, C)], idx_v.at[b], sem_i.at[b]
            )
            idx_cp[i].start()

        start_idx(0)
        for i in range(nchunks):
            b = i % 2
            if i + 1 < nchunks:
                start_idx(i + 1)
            idx_cp[i].wait()
            if i >= 2:
                st_cp[i - 2].wait()
            gat_cp[i] = pltpu.make_async_copy(
                table_hbm.at[idx_v.at[b]], rows_v.at[b], sem_g.at[b]
            )
            gat_cp[i].start()
            gat_cp[i].wait()
            st_cp[i] = pltpu.make_async_copy(
                rows_v.at[b], out_hbm.at[pl.ds(base + i * C, C)], sem_s.at[b]
            )
            st_cp[i].start()
        st_cp[nchunks - 2].wait()
        st_cp[nchunks - 1].wait()

    return k(idx, weight)


def kernel(input, weight):
    N, W = input.shape
    D = weight.shape[1]
    B = N * W
    idx = input.T.reshape(B).astype(jnp.int32)
    table = jax.lax.optimization_barrier(weight.reshape(-1, 4 * D))
    table = table.reshape(-1, D)
    out = _embedding_gather(idx, table, B, D)
    return out.reshape(W, N, D).transpose(1, 0, 2)
